# Initial kernel scaffold; baseline (speedup 1.0000x reference)
import jax
import jax.numpy as jnp
from jax.experimental import pallas as pl


def kernel(embeds, embed_type, table):
    raise NotImplementedError("write your pallas kernel here")


# speedup vs baseline: 3.7313x; 3.7313x over previous
import jax
import jax.numpy as jnp
from jax.experimental import pallas as pl
from jax.experimental.pallas import tpu as pltpu

_BM = 512


def _add_row_kernel(idx_ref, table_ref, x_ref, o_ref):
    row = table_ref[idx_ref[0], :]
    o_ref[...] = x_ref[...] + row[None, :]


def kernel(embeds, embed_type, table):
    b, s, h = embeds.shape
    n = b * s
    x = embeds.reshape(n, h)
    idx = jnp.asarray(embed_type, dtype=jnp.int32).reshape(1)
    out = pl.pallas_call(
        _add_row_kernel,
        grid_spec=pltpu.PrefetchScalarGridSpec(
            num_scalar_prefetch=1,
            grid=(n // _BM,),
            in_specs=[
                pl.BlockSpec(table.shape, lambda i, idx_ref: (0, 0)),
                pl.BlockSpec((_BM, h), lambda i, idx_ref: (i, 0)),
            ],
            out_specs=pl.BlockSpec((_BM, h), lambda i, idx_ref: (i, 0)),
        ),
        out_shape=jax.ShapeDtypeStruct((n, h), embeds.dtype),
    )(idx, table, x)
    return out.reshape(b, s, h)


# speedup vs baseline: 3.7346x; 1.0009x over previous
import jax
import jax.numpy as jnp
from jax.experimental import pallas as pl
from jax.experimental.pallas import tpu as pltpu

_BM = 512


def _add_row_kernel(idx_ref, table_ref, x_ref, o_ref):
    row = table_ref[idx_ref[0], :]
    o_ref[...] = x_ref[...] + row[None, :]


def kernel(embeds, embed_type, table):
    b, s, h = embeds.shape
    n = b * s
    x = embeds.reshape(n, h)
    idx = jnp.asarray(embed_type, dtype=jnp.int32).reshape(1)
    out = pl.pallas_call(
        _add_row_kernel,
        grid_spec=pltpu.PrefetchScalarGridSpec(
            num_scalar_prefetch=1,
            grid=(n // _BM,),
            in_specs=[
                pl.BlockSpec(table.shape, lambda i, idx_ref: (0, 0)),
                pl.BlockSpec((_BM, h), lambda i, idx_ref: (i, 0)),
            ],
            out_specs=pl.BlockSpec((_BM, h), lambda i, idx_ref: (i, 0)),
        ),
        out_shape=jax.ShapeDtypeStruct((n, h), embeds.dtype),
        compiler_params=pltpu.CompilerParams(
            dimension_semantics=("parallel",),
        ),
    )(idx, table, x)
    return out.reshape(b, s, h)


# speedup vs baseline: 4.0580x; 1.0866x over previous
import jax
import jax.numpy as jnp
from jax.experimental import pallas as pl
from jax.experimental.pallas import tpu as pltpu

_BM = 1024


def _add_row_kernel(idx_ref, table_ref, x_ref, o_ref):
    row = table_ref[idx_ref[0], :]
    o_ref[...] = x_ref[...] + row[None, :]


def kernel(embeds, embed_type, table):
    b, s, h = embeds.shape
    n = b * s
    x = embeds.reshape(n, h)
    idx = jnp.asarray(embed_type, dtype=jnp.int32).reshape(1)
    out = pl.pallas_call(
        _add_row_kernel,
        grid_spec=pltpu.PrefetchScalarGridSpec(
            num_scalar_prefetch=1,
            grid=(n // _BM,),
            in_specs=[
                pl.BlockSpec(table.shape, lambda i, idx_ref: (0, 0)),
                pl.BlockSpec((_BM, h), lambda i, idx_ref: (i, 0)),
            ],
            out_specs=pl.BlockSpec((_BM, h), lambda i, idx_ref: (i, 0)),
        ),
        out_shape=jax.ShapeDtypeStruct((n, h), embeds.dtype),
        compiler_params=pltpu.CompilerParams(
            dimension_semantics=("parallel",),
        ),
    )(idx, table, x)
    return out.reshape(b, s, h)


# speedup vs baseline: 4.2128x; 1.0382x over previous
import jax
import jax.numpy as jnp
from jax.experimental import pallas as pl
from jax.experimental.pallas import tpu as pltpu

_BM = 2048


def _add_row_kernel(idx_ref, table_ref, x_ref, o_ref):
    row = table_ref[idx_ref[0], :]
    o_ref[...] = x_ref[...] + row[None, :]


def kernel(embeds, embed_type, table):
    b, s, h = embeds.shape
    n = b * s
    x = embeds.reshape(n, h)
    idx = jnp.asarray(embed_type, dtype=jnp.int32).reshape(1)
    out = pl.pallas_call(
        _add_row_kernel,
        grid_spec=pltpu.PrefetchScalarGridSpec(
            num_scalar_prefetch=1,
            grid=(n // _BM,),
            in_specs=[
                pl.BlockSpec(table.shape, lambda i, idx_ref: (0, 0)),
                pl.BlockSpec((_BM, h), lambda i, idx_ref: (i, 0)),
            ],
            out_specs=pl.BlockSpec((_BM, h), lambda i, idx_ref: (i, 0)),
        ),
        out_shape=jax.ShapeDtypeStruct((n, h), embeds.dtype),
        compiler_params=pltpu.CompilerParams(
            dimension_semantics=("parallel",),
        ),
    )(idx, table, x)
    return out.reshape(b, s, h)
